# Initial kernel scaffold; baseline (speedup 1.0000x reference)
import functools

import jax
import jax.numpy as jnp
from jax import lax
from jax.experimental import pallas as pl
from jax.experimental.pallas import tpu as pltpu
from jax.experimental.pallas import tpu_sc as plsc

N = 10000
E = 160000
IN = 256
H = 512
OUT = 256
L = 4
EPS = 0.1

NP = 10240
C = 4
CW = 128
NC, NS = 2, 16
EB = 128
RPW = 84
RB = NS * RPW
EP = RB * EB
PAD_NODE = NP - 1
ROWS_PER_SUB = NP // NS
RBLK = 1280
GRID = NP // RBLK

_mesh = plsc.VectorSubcoreMesh(core_axis_name="c", subcore_axis_name="s")


def _zero_vmem(ref, nrows):
    def body(i, _):
        for j in range(ref.shape[1] // 16):
            ref[i, pl.ds(j * 16, 16)] = jnp.zeros((16,), jnp.float32)
        return 0

    lax.fori_loop(0, nrows, body, 0)


@functools.partial(
    pl.kernel,
    out_type=jax.ShapeDtypeStruct((NP, 16), jnp.float32),
    mesh=_mesh,
    scratch_types=[
        pltpu.VMEM((RPW, EB), jnp.int32),
        pltpu.VMEM((128, 16), jnp.float32),
        pltpu.VMEM((128, 16), jnp.float32),
        pltpu.VMEM_SHARED((NP, 16), jnp.float32),
    ],
)
def _sc_degree(dst_hbm, deg_out, dstv, onesv, zv, acc):
    cc = lax.axis_index("c")
    sid = lax.axis_index("s")

    @pl.when(cc == 0)
    def _():
        lane = lax.iota(jnp.int32, 16)
        e2 = jnp.where(lane == 2, 1.0, 0.0).astype(jnp.float32)

        def fill(i, _):
            onesv[i, :] = e2
            zv[i, :] = jnp.zeros((16,), jnp.float32)
            return 0

        lax.fori_loop(0, 128, fill, 0)
        for k in range(ROWS_PER_SUB // 128):
            pltpu.sync_copy(zv, acc.at[pl.ds(sid * ROWS_PER_SUB + k * 128, 128)])
        plsc.subcore_barrier()
        pltpu.sync_copy(dst_hbm.at[pl.ds(sid * RPW, RPW)], dstv)

        def batch(b, _):
            pltpu.sync_copy(onesv, acc.at[dstv.at[b]], add=True)
            return 0

        lax.fori_loop(0, RPW, batch, 0)
        plsc.subcore_barrier()
        for k in range(ROWS_PER_SUB // 128):
            sl = pl.ds(sid * ROWS_PER_SUB + k * 128, 128)
            pltpu.sync_copy(acc.at[sl], deg_out.at[sl])


@functools.partial(
    pl.kernel,
    out_type=[jax.ShapeDtypeStruct((NP, CW), jnp.float32) for _ in range(C)],
    mesh=_mesh,
    scratch_types=[
        pltpu.VMEM((RPW, EB), jnp.int32),
        pltpu.VMEM((RPW, EB), jnp.int32),
        pltpu.VMEM((RPW, EB), jnp.float32),
        pltpu.VMEM((EB, 16), jnp.float32),
        pltpu.VMEM((EB, 16), jnp.float32),
        pltpu.VMEM((EB, CW), jnp.float32),
        pltpu.VMEM((128, CW), jnp.float32),
        pltpu.VMEM_SHARED((NP, CW), jnp.float32),
    ],
)
def _sc_spmm(src_hbm, dst_hbm, t_hbm, h0b, h1b, h2b, h3b,
             o0, o1, o2, o3, srcv, dstv, wv, sv, dv, rows, zbuf, acc):
    cc = lax.axis_index("c")
    sid = lax.axis_index("s")
    base = sid * RPW
    pltpu.sync_copy(src_hbm.at[pl.ds(base, RPW)], srcv)
    pltpu.sync_copy(dst_hbm.at[pl.ds(base, RPW)], dstv)

    lane = lax.iota(jnp.int32, 16)
    c0 = jnp.zeros((16,), jnp.int32)
    c1 = c0 + 1
    c2 = c0 + 2

    def wbatch(b, _):
        pltpu.sync_copy(t_hbm.at[srcv.at[b]], sv)
        pltpu.sync_copy(t_hbm.at[dstv.at[b]], dv)
        for g in range(EB // 16):
            idx = lane + g * 16
            al_s = plsc.load_gather(sv, [idx, c0])
            ar_d = plsc.load_gather(dv, [idx, c1])
            di_s = plsc.load_gather(sv, [idx, c2])
            di_d = plsc.load_gather(dv, [idx, c2])
            a = al_s + ar_d
            e = jnp.exp(jnp.abs(a) * -2.0)
            t = (1.0 - e) / (1.0 + e)
            wv[b, pl.ds(g * 16, 16)] = di_s * di_d * jnp.sign(a) * t
        return 0

    lax.fori_loop(0, RPW, wbatch, 0)
    _zero_vmem(zbuf, 128)

    def chunk(h_hbm, o_hbm):
        for k in range(ROWS_PER_SUB // 128):
            pltpu.sync_copy(zbuf, acc.at[pl.ds(sid * ROWS_PER_SUB + k * 128, 128)])
        plsc.subcore_barrier()

        def ebatch(b, _):
            pltpu.sync_copy(h_hbm.at[srcv.at[b]], rows)

            def srow(i, _):
                wi = wv[b, i]
                for j in range(CW // 16):
                    sl = rows[i, pl.ds(j * 16, 16)]
                    rows[i, pl.ds(j * 16, 16)] = sl * wi
                return 0

            lax.fori_loop(0, EB, srow, 0)
            pltpu.sync_copy(rows, acc.at[dstv.at[b]], add=True)
            return 0

        lax.fori_loop(0, RPW, ebatch, 0)
        plsc.subcore_barrier()
        for k in range(ROWS_PER_SUB // 128):
            sl = pl.ds(sid * ROWS_PER_SUB + k * 128, 128)
            pltpu.sync_copy(acc.at[sl], o_hbm.at[sl])
        plsc.subcore_barrier()

    @pl.when(cc == 0)
    def _():
        chunk(h0b, o0)
        chunk(h1b, o1)

    @pl.when(cc == 1)
    def _():
        chunk(h2b, o2)
        chunk(h3b, o3)


def _dinv16(deg):
    return jnp.where(deg > 0, lax.rsqrt(jnp.maximum(deg, 1e-30)), 0.0)


def _tc_prep_body(x_ref, w_ref, b_ref, deg_ref, att_ref,
                  h0, h1, h2, h3, t_ref):
    h = jnp.dot(x_ref[...], w_ref[...], preferred_element_type=jnp.float32)
    h = h + b_ref[...]
    for cix, hr in enumerate((h0, h1, h2, h3)):
        hr[...] = h[:, cix * CW:(cix + 1) * CW]
    t_ref[...] = (jnp.dot(h, att_ref[...], preferred_element_type=jnp.float32)
                  + _dinv16(deg_ref[...]))


def _tc_mid_body(o0, o1, o2, o3, p0, p1, p2, p3, att_ref, deg_ref,
                 h0, h1, h2, h3, t_ref):
    hcs = []
    for o, p, hr in ((o0, p0, h0), (o1, p1, h1), (o2, p2, h2), (o3, p3, h3)):
        hc = o[...] + EPS * p[...]
        hr[...] = hc
        hcs.append(hc)
    h = jnp.concatenate(hcs, axis=1)
    t_ref[...] = (jnp.dot(h, att_ref[...], preferred_element_type=jnp.float32)
                  + _dinv16(deg_ref[...]))


def _tc_final_body(o0, o1, o2, o3, p0, p1, p2, p3, w_ref, b_ref, y_ref):
    h = jnp.concatenate(
        [o[...] + EPS * p[...] for o, p in ((o0, p0), (o1, p1), (o2, p2), (o3, p3))],
        axis=1)
    y_ref[...] = (jnp.dot(h, w_ref[...], preferred_element_type=jnp.float32)
                  + b_ref[...])


def _row_spec(d):
    return pl.BlockSpec((RBLK, d), lambda i: (i, 0))


def _full_spec(r, d):
    return pl.BlockSpec((r, d), lambda i: (0, 0))


_tc_prep = pl.pallas_call(
    _tc_prep_body,
    grid=(GRID,),
    in_specs=[_row_spec(IN), _full_spec(IN, H), _full_spec(1, H),
              _row_spec(16), _full_spec(H, 16)],
    out_specs=[_row_spec(CW)] * C + [_row_spec(16)],
    out_shape=[jax.ShapeDtypeStruct((NP, CW), jnp.float32) for _ in range(C)]
    + [jax.ShapeDtypeStruct((NP, 16), jnp.float32)],
)

_tc_mid = pl.pallas_call(
    _tc_mid_body,
    grid=(GRID,),
    in_specs=[_row_spec(CW)] * (2 * C) + [_full_spec(H, 16), _row_spec(16)],
    out_specs=[_row_spec(CW)] * C + [_row_spec(16)],
    out_shape=[jax.ShapeDtypeStruct((NP, CW), jnp.float32) for _ in range(C)]
    + [jax.ShapeDtypeStruct((NP, 16), jnp.float32)],
)

_tc_final = pl.pallas_call(
    _tc_final_body,
    grid=(GRID,),
    in_specs=[_row_spec(CW)] * (2 * C) + [_full_spec(H, OUT), _full_spec(1, OUT)],
    out_specs=_row_spec(OUT),
    out_shape=jax.ShapeDtypeStruct((NP, OUT), jnp.float32),
)


def kernel(x, edge_index, W_in, b_in, att_l, att_r, W_out, b_out):
    loop = jnp.arange(N, dtype=jnp.int32)
    src = jnp.concatenate([edge_index[0], loop])
    dst = jnp.concatenate([edge_index[1], loop])
    pad = jnp.full((EP - E - N,), PAD_NODE, dtype=jnp.int32)
    src2d = jnp.concatenate([src, pad]).reshape(RB, EB)
    dst2d = jnp.concatenate([dst, pad]).reshape(RB, EB)
    xp = jnp.pad(x, ((0, NP - N), (0, 0)))
    att_big = [
        jnp.zeros((H, 16), jnp.float32)
        .at[:, 0].set(att_l[l]).at[:, 1].set(att_r[l])
        for l in range(L)
    ]
    b_in2 = b_in[None, :]
    b_out2 = b_out[None, :]

    deg16 = _sc_degree(dst2d)
    *h0c, t = _tc_prep(xp, W_in, b_in2, deg16, att_big[0])
    hc = list(h0c)
    oc = None
    for l in range(L):
        oc = _sc_spmm(src2d, dst2d, t, *hc)
        if l + 1 < L:
            *hc, t = _tc_mid(*oc, *h0c, att_big[l + 1], deg16)
    y = _tc_final(*oc, *h0c, W_out, b_out2)
    return y[:N]


# speedup vs baseline: 2.7922x; 2.7922x over previous
import functools

import jax
import jax.numpy as jnp
from jax import lax
from jax.experimental import pallas as pl
from jax.experimental.pallas import tpu as pltpu
from jax.experimental.pallas import tpu_sc as plsc

N = 10000
E = 160000
IN = 256
H = 512
OUT = 256
L = 4
EPS = 0.1

NP = 10240
C = 8
CW = 64
NC, NS = 2, 16
EB = 128
RPW = 88
RB = NS * RPW
EP = RB * EB
PAD_NODE = NP - 1
ROWS_PER_SUB = NP // NS
RBLK = 1280
GRID = NP // RBLK

_mesh = plsc.VectorSubcoreMesh(core_axis_name="c", subcore_axis_name="s")


def _zero_vmem(ref, nrows):
    def body(i, _):
        for j in range(ref.shape[1] // 16):
            ref[i, pl.ds(j * 16, 16)] = jnp.zeros((16,), jnp.float32)
        return 0

    lax.fori_loop(0, nrows, body, 0)


@functools.partial(
    pl.kernel,
    out_type=jax.ShapeDtypeStruct((NP, 16), jnp.float32),
    mesh=_mesh,
    scratch_types=[
        pltpu.VMEM((RPW, EB), jnp.int32),
        pltpu.VMEM((128, 16), jnp.float32),
        pltpu.VMEM((128, 16), jnp.float32),
        pltpu.VMEM_SHARED((NP, 16), jnp.float32),
    ],
    compiler_params=pltpu.CompilerParams(use_tc_tiling_on_sc=False),
)
def _sc_degree(dst_hbm, deg_out, dstv, onesv, zv, acc):
    cc = lax.axis_index("c")
    sid = lax.axis_index("s")

    @pl.when(cc == 0)
    def _():
        lane = lax.iota(jnp.int32, 16)
        e2 = jnp.where(lane == 2, 1.0, 0.0).astype(jnp.float32)

        def fill(i, _):
            onesv[i, :] = e2
            zv[i, :] = jnp.zeros((16,), jnp.float32)
            return 0

        lax.fori_loop(0, 128, fill, 0)
        for k in range(ROWS_PER_SUB // 128):
            pltpu.sync_copy(zv, acc.at[pl.ds(sid * ROWS_PER_SUB + k * 128, 128)])
        plsc.subcore_barrier()
        pltpu.sync_copy(dst_hbm.at[pl.ds(sid * RPW, RPW)], dstv)

        def batch(b, _):
            pltpu.sync_copy(onesv, acc.at[dstv.at[b]], add=True)
            return 0

        lax.fori_loop(0, RPW, batch, 0)
        plsc.subcore_barrier()
        for k in range(ROWS_PER_SUB // 128):
            sl = pl.ds(sid * ROWS_PER_SUB + k * 128, 128)
            pltpu.sync_copy(acc.at[sl], deg_out.at[sl])


@functools.partial(
    pl.kernel,
    out_type=[jax.ShapeDtypeStruct((NP, CW), jnp.float32) for _ in range(C)],
    mesh=_mesh,
    scratch_types=[
        pltpu.VMEM((RPW, EB), jnp.int32),
        pltpu.VMEM((RPW, EB), jnp.int32),
        pltpu.VMEM((RPW, EB), jnp.float32),
        pltpu.VMEM((EB,), jnp.float32),
        pltpu.VMEM((EB,), jnp.float32),
        pltpu.VMEM((EB,), jnp.float32),
        pltpu.VMEM((EB,), jnp.float32),
        pltpu.VMEM((EB, CW), jnp.float32),
        pltpu.VMEM((128, CW), jnp.float32),
        pltpu.VMEM_SHARED((NP, CW), jnp.float32),
    ],
    compiler_params=pltpu.CompilerParams(use_tc_tiling_on_sc=False),
)
def _sc_spmm(src_hbm, dst_hbm, al_hbm, ar_hbm, di_hbm,
             h0b, h1b, h2b, h3b, h4b, h5b, h6b, h7b,
             o0, o1, o2, o3, o4, o5, o6, o7, srcv, dstv, wv, als, ard, dis, did,
             rows, zbuf, acc):
    cc = lax.axis_index("c")
    sid = lax.axis_index("s")
    base = sid * RPW
    pltpu.sync_copy(src_hbm.at[pl.ds(base, RPW)], srcv)
    pltpu.sync_copy(dst_hbm.at[pl.ds(base, RPW)], dstv)

    def wbatch(b, _):
        pltpu.sync_copy(al_hbm.at[srcv.at[b]], als)
        pltpu.sync_copy(ar_hbm.at[dstv.at[b]], ard)
        pltpu.sync_copy(di_hbm.at[srcv.at[b]], dis)
        pltpu.sync_copy(di_hbm.at[dstv.at[b]], did)
        for g in range(EB // 16):
            sl = pl.ds(g * 16, 16)
            a = als[sl] + ard[sl]
            e = jnp.exp(jnp.abs(a) * -2.0)
            t = (1.0 - e) / (1.0 + e)
            wv[b, sl] = dis[sl] * did[sl] * jnp.sign(a) * t
        return 0

    lax.fori_loop(0, RPW, wbatch, 0)
    _zero_vmem(zbuf, 128)

    def chunk(h_hbm, o_hbm):
        for k in range(ROWS_PER_SUB // 128):
            pltpu.sync_copy(zbuf, acc.at[pl.ds(sid * ROWS_PER_SUB + k * 128, 128)])
        plsc.subcore_barrier()

        def ebatch(b, _):
            pltpu.sync_copy(h_hbm.at[srcv.at[b]], rows)

            def sgroup(g, _):
                w16 = wv[b, pl.ds(g * 16, 16)]
                for k in range(16):
                    i = g * 16 + k
                    wi = w16[k]
                    for j in range(CW // 16):
                        sl = rows[i, pl.ds(j * 16, 16)]
                        rows[i, pl.ds(j * 16, 16)] = sl * wi
                return 0

            lax.fori_loop(0, EB // 16, sgroup, 0)
            pltpu.sync_copy(rows, acc.at[dstv.at[b]], add=True)
            return 0

        lax.fori_loop(0, RPW, ebatch, 0)
        plsc.subcore_barrier()
        for k in range(ROWS_PER_SUB // 128):
            sl = pl.ds(sid * ROWS_PER_SUB + k * 128, 128)
            pltpu.sync_copy(acc.at[sl], o_hbm.at[sl])
        plsc.subcore_barrier()

    @pl.when(cc == 0)
    def _():
        chunk(h0b, o0)
        chunk(h1b, o1)
        chunk(h2b, o2)
        chunk(h3b, o3)

    @pl.when(cc == 1)
    def _():
        chunk(h4b, o4)
        chunk(h5b, o5)
        chunk(h6b, o6)
        chunk(h7b, o7)


def _dinv16(deg):
    return jnp.where(deg > 0, lax.rsqrt(jnp.maximum(deg, 1e-30)), 0.0)


def _tc_prep_body(x_ref, w_ref, b_ref, deg_ref, att_ref, *rest):
    hrefs, t_ref = rest[:C], rest[C]
    h = jnp.dot(x_ref[...], w_ref[...], preferred_element_type=jnp.float32)
    h = h + b_ref[...]
    for cix, hr in enumerate(hrefs):
        hr[...] = h[:, cix * CW:(cix + 1) * CW]
    t_ref[...] = (jnp.dot(h, att_ref[...], preferred_element_type=jnp.float32)
                  + _dinv16(deg_ref[...]))


def _tc_mid_body(*refs):
    ocs, pcs = refs[:C], refs[C:2 * C]
    att_ref, deg_ref = refs[2 * C], refs[2 * C + 1]
    hrefs, t_ref = refs[2 * C + 2:3 * C + 2], refs[3 * C + 2]
    hcs = []
    for o, p, hr in zip(ocs, pcs, hrefs):
        hc = o[...] + EPS * p[...]
        hr[...] = hc
        hcs.append(hc)
    h = jnp.concatenate(hcs, axis=1)
    t_ref[...] = (jnp.dot(h, att_ref[...], preferred_element_type=jnp.float32)
                  + _dinv16(deg_ref[...]))


def _tc_final_body(*refs):
    ocs, pcs = refs[:C], refs[C:2 * C]
    w_ref, b_ref, y_ref = refs[2 * C], refs[2 * C + 1], refs[2 * C + 2]
    h = jnp.concatenate(
        [o[...] + EPS * p[...] for o, p in zip(ocs, pcs)], axis=1)
    y_ref[...] = (jnp.dot(h, w_ref[...], preferred_element_type=jnp.float32)
                  + b_ref[...])


def _row_spec(d):
    return pl.BlockSpec((RBLK, d), lambda i: (i, 0))


def _full_spec(r, d):
    return pl.BlockSpec((r, d), lambda i: (0, 0))


_tc_prep = pl.pallas_call(
    _tc_prep_body,
    grid=(GRID,),
    in_specs=[_row_spec(IN), _full_spec(IN, H), _full_spec(1, H),
              _row_spec(16), _full_spec(H, 16)],
    out_specs=[_row_spec(CW)] * C + [_row_spec(16)],
    out_shape=[jax.ShapeDtypeStruct((NP, CW), jnp.float32) for _ in range(C)]
    + [jax.ShapeDtypeStruct((NP, 16), jnp.float32)],
)

_tc_mid = pl.pallas_call(
    _tc_mid_body,
    grid=(GRID,),
    in_specs=[_row_spec(CW)] * (2 * C) + [_full_spec(H, 16), _row_spec(16)],
    out_specs=[_row_spec(CW)] * C + [_row_spec(16)],
    out_shape=[jax.ShapeDtypeStruct((NP, CW), jnp.float32) for _ in range(C)]
    + [jax.ShapeDtypeStruct((NP, 16), jnp.float32)],
)

_tc_final = pl.pallas_call(
    _tc_final_body,
    grid=(GRID,),
    in_specs=[_row_spec(CW)] * (2 * C) + [_full_spec(H, OUT), _full_spec(1, OUT)],
    out_specs=_row_spec(OUT),
    out_shape=jax.ShapeDtypeStruct((NP, OUT), jnp.float32),
)


def kernel(x, edge_index, W_in, b_in, att_l, att_r, W_out, b_out):
    loop = jnp.arange(N, dtype=jnp.int32)
    src = jnp.concatenate([edge_index[0], loop])
    dst = jnp.concatenate([edge_index[1], loop])
    pad = jnp.full((EP - E - N,), PAD_NODE, dtype=jnp.int32)
    src2d = jnp.concatenate([src, pad]).reshape(RB, EB)
    dst2d = jnp.concatenate([dst, pad]).reshape(RB, EB)
    xp = jnp.pad(x, ((0, NP - N), (0, 0)))
    att_big = [
        jnp.zeros((H, 16), jnp.float32)
        .at[:, 0].set(att_l[l]).at[:, 1].set(att_r[l])
        for l in range(L)
    ]
    b_in2 = b_in[None, :]
    b_out2 = b_out[None, :]

    deg16 = _sc_degree(dst2d)
    *h0c, t = _tc_prep(xp, W_in, b_in2, deg16, att_big[0])
    hc = list(h0c)
    oc = None
    for l in range(L):
        al1, ar1, di1 = t[:, 0], t[:, 1], t[:, 2]
        oc = _sc_spmm(src2d, dst2d, al1, ar1, di1, *hc)
        if l + 1 < L:
            *hc, t = _tc_mid(*oc, *h0c, att_big[l + 1], deg16)
    y = _tc_final(*oc, *h0c, W_out, b_out2)
    return y[:N]


# speedup vs baseline: 3.5500x; 1.2714x over previous
import functools

import jax
import jax.numpy as jnp
from jax import lax
from jax.experimental import pallas as pl
from jax.experimental.pallas import tpu as pltpu
from jax.experimental.pallas import tpu_sc as plsc

N = 10000
E = 160000
IN = 256
H = 512
OUT = 256
L = 4
EPS = 0.1

NP = 10240
C = 8
CW = 64
NC, NS = 2, 16
EB = 128
RPW = 88
RB = NS * RPW
EP = RB * EB
PAD_NODE = NP - 1
ROWS_PER_SUB = NP // NS
RBLK = 1280
GRID = NP // RBLK

_mesh = plsc.VectorSubcoreMesh(core_axis_name="c", subcore_axis_name="s")


def _zero_vmem(ref, nrows):
    def body(i, _):
        for j in range(ref.shape[1] // 16):
            ref[i, pl.ds(j * 16, 16)] = jnp.zeros((16,), jnp.float32)
        return 0

    lax.fori_loop(0, nrows, body, 0)


@functools.partial(
    pl.kernel,
    out_type=jax.ShapeDtypeStruct((NP, 16), jnp.float32),
    mesh=_mesh,
    scratch_types=[
        pltpu.VMEM((RPW, EB), jnp.int32),
        pltpu.VMEM((128, 16), jnp.float32),
        pltpu.VMEM((128, 16), jnp.float32),
        pltpu.VMEM_SHARED((NP, 16), jnp.float32),
    ],
    compiler_params=pltpu.CompilerParams(use_tc_tiling_on_sc=False),
)
def _sc_degree(dst_hbm, deg_out, dstv, onesv, zv, acc):
    cc = lax.axis_index("c")
    sid = lax.axis_index("s")

    @pl.when(cc == 0)
    def _():
        lane = lax.iota(jnp.int32, 16)
        e2 = jnp.where(lane == 2, 1.0, 0.0).astype(jnp.float32)

        def fill(i, _):
            onesv[i, :] = e2
            zv[i, :] = jnp.zeros((16,), jnp.float32)
            return 0

        lax.fori_loop(0, 128, fill, 0)
        for k in range(ROWS_PER_SUB // 128):
            pltpu.sync_copy(zv, acc.at[pl.ds(sid * ROWS_PER_SUB + k * 128, 128)])
        plsc.subcore_barrier()
        pltpu.sync_copy(dst_hbm.at[pl.ds(sid * RPW, RPW)], dstv)

        def batch(b, _):
            pltpu.sync_copy(onesv, acc.at[dstv.at[b]], add=True)
            return 0

        lax.fori_loop(0, RPW, batch, 0)
        plsc.subcore_barrier()
        for k in range(ROWS_PER_SUB // 128):
            sl = pl.ds(sid * ROWS_PER_SUB + k * 128, 128)
            pltpu.sync_copy(acc.at[sl], deg_out.at[sl])


@functools.partial(
    pl.kernel,
    out_type=[jax.ShapeDtypeStruct((NP, CW), jnp.float32) for _ in range(C)],
    mesh=_mesh,
    scratch_types=[
        pltpu.VMEM((RPW, EB), jnp.int32),
        pltpu.VMEM((RPW, EB), jnp.int32),
        pltpu.VMEM((RPW, EB), jnp.float32),
        pltpu.VMEM((2, EB), jnp.float32),
        pltpu.VMEM((2, EB), jnp.float32),
        pltpu.VMEM((2, EB), jnp.float32),
        pltpu.VMEM((2, EB), jnp.float32),
        pltpu.VMEM((2, EB, CW), jnp.float32),
        pltpu.VMEM((128, CW), jnp.float32),
        pltpu.VMEM_SHARED((NP, CW), jnp.float32),
        pltpu.SemaphoreType.DMA,
        pltpu.SemaphoreType.DMA,
        pltpu.SemaphoreType.DMA,
        pltpu.SemaphoreType.DMA,
    ],
    compiler_params=pltpu.CompilerParams(use_tc_tiling_on_sc=False),
)
def _sc_spmm(src_hbm, dst_hbm, al_hbm, ar_hbm, di_hbm,
             h0b, h1b, h2b, h3b, h4b, h5b, h6b, h7b,
             o0, o1, o2, o3, o4, o5, o6, o7,
             srcv, dstv, wv, als, ard, dis, did,
             rows, zbuf, acc, gsem, ssem, wsem, rsem):
    cc = lax.axis_index("c")
    sid = lax.axis_index("s")
    base = sid * RPW
    pltpu.sync_copy(src_hbm.at[pl.ds(base, RPW)], srcv)
    pltpu.sync_copy(dst_hbm.at[pl.ds(base, RPW)], dstv)
    _zero_vmem(zbuf, 128)

    def fire_w(b, p):
        pltpu.async_copy(al_hbm.at[srcv.at[b]], als.at[p], wsem)
        pltpu.async_copy(ar_hbm.at[dstv.at[b]], ard.at[p], wsem)
        pltpu.async_copy(di_hbm.at[srcv.at[b]], dis.at[p], wsem)
        pltpu.async_copy(di_hbm.at[dstv.at[b]], did.at[p], wsem)

    def wait_w(b, p):
        pltpu.make_async_copy(al_hbm.at[srcv.at[b]], als.at[p], wsem).wait()
        pltpu.make_async_copy(ar_hbm.at[dstv.at[b]], ard.at[p], wsem).wait()
        pltpu.make_async_copy(di_hbm.at[srcv.at[b]], dis.at[p], wsem).wait()
        pltpu.make_async_copy(di_hbm.at[dstv.at[b]], did.at[p], wsem).wait()

    def chunk(h_hbm, o_hbm, first):
        for k in range(ROWS_PER_SUB // 128):
            pltpu.sync_copy(zbuf, acc.at[pl.ds(sid * ROWS_PER_SUB + k * 128, 128)])
        plsc.subcore_barrier()

        pltpu.async_copy(h_hbm.at[srcv.at[0]], rows.at[0], gsem)
        if first:
            fire_w(0, 0)

        def outer(g, _):
            for par in range(2):
                b = g * 2 + par
                buf = rows.at[par]
                pltpu.make_async_copy(h_hbm.at[srcv.at[b]], buf, gsem).wait()
                if first:
                    wait_w(b, par)

                    @pl.when(b + 1 < RPW)
                    def _():
                        fire_w(b + 1, 1 - par)

                    for gq in range(EB // 16):
                        sl = pl.ds(gq * 16, 16)
                        a = als[par, sl] + ard[par, sl]
                        e = jnp.exp(jnp.abs(a) * -2.0)
                        t = (1.0 - e) / (1.0 + e)
                        wv[b, sl] = dis[par, sl] * did[par, sl] * jnp.sign(a) * t

                def sgroup(gq, _):
                    w16 = wv[b, pl.ds(gq * 16, 16)]
                    for k in range(16):
                        i = gq * 16 + k
                        wi = w16[k]
                        for j in range(CW // 16):
                            sl = pl.ds(j * 16, 16)
                            buf[i, sl] = buf[i, sl] * wi
                    return 0

                lax.fori_loop(0, EB // 16, sgroup, 0)

                @pl.when(b > 0)
                def _():
                    pltpu.make_async_copy(
                        rows.at[1 - par], acc.at[dstv.at[b - 1]], ssem).wait()

                pltpu.async_copy(buf, acc.at[dstv.at[b]], ssem, add=True)

                @pl.when(b + 1 < RPW)
                def _():
                    pltpu.async_copy(h_hbm.at[srcv.at[b + 1]], rows.at[1 - par], gsem)
            return 0

        lax.fori_loop(0, RPW // 2, outer, 0)
        pltpu.make_async_copy(rows.at[1], acc.at[dstv.at[RPW - 1]], ssem).wait()
        plsc.subcore_barrier()
        for k in range(ROWS_PER_SUB // 128):
            sl = pl.ds(sid * ROWS_PER_SUB + k * 128, 128)
            pltpu.async_copy(acc.at[sl], o_hbm.at[sl], rsem)
        for k in range(ROWS_PER_SUB // 128):
            sl = pl.ds(sid * ROWS_PER_SUB + k * 128, 128)
            pltpu.make_async_copy(acc.at[sl], o_hbm.at[sl], rsem).wait()
        plsc.subcore_barrier()

    @pl.when(cc == 0)
    def _():
        chunk(h0b, o0, True)
        chunk(h1b, o1, False)
        chunk(h2b, o2, False)
        chunk(h3b, o3, False)

    @pl.when(cc == 1)
    def _():
        chunk(h4b, o4, True)
        chunk(h5b, o5, False)
        chunk(h6b, o6, False)
        chunk(h7b, o7, False)


def _dinv16(deg):
    return jnp.where(deg > 0, lax.rsqrt(jnp.maximum(deg, 1e-30)), 0.0)


def _tc_prep_body(x_ref, w_ref, b_ref, deg_ref, att_ref, *rest):
    hrefs, t_ref = rest[:C], rest[C]
    h = jnp.dot(x_ref[...], w_ref[...], preferred_element_type=jnp.float32)
    h = h + b_ref[...]
    for cix, hr in enumerate(hrefs):
        hr[...] = h[:, cix * CW:(cix + 1) * CW]
    t_ref[...] = (jnp.dot(h, att_ref[...], preferred_element_type=jnp.float32)
                  + _dinv16(deg_ref[...]))


def _tc_mid_body(*refs):
    ocs, pcs = refs[:C], refs[C:2 * C]
    att_ref, deg_ref = refs[2 * C], refs[2 * C + 1]
    hrefs, t_ref = refs[2 * C + 2:3 * C + 2], refs[3 * C + 2]
    hcs = []
    for o, p, hr in zip(ocs, pcs, hrefs):
        hc = o[...] + EPS * p[...]
        hr[...] = hc
        hcs.append(hc)
    h = jnp.concatenate(hcs, axis=1)
    t_ref[...] = (jnp.dot(h, att_ref[...], preferred_element_type=jnp.float32)
                  + _dinv16(deg_ref[...]))


def _tc_final_body(*refs):
    ocs, pcs = refs[:C], refs[C:2 * C]
    w_ref, b_ref, y_ref = refs[2 * C], refs[2 * C + 1], refs[2 * C + 2]
    h = jnp.concatenate(
        [o[...] + EPS * p[...] for o, p in zip(ocs, pcs)], axis=1)
    y_ref[...] = (jnp.dot(h, w_ref[...], preferred_element_type=jnp.float32)
                  + b_ref[...])


def _row_spec(d):
    return pl.BlockSpec((RBLK, d), lambda i: (i, 0))


def _full_spec(r, d):
    return pl.BlockSpec((r, d), lambda i: (0, 0))


_tc_prep = pl.pallas_call(
    _tc_prep_body,
    grid=(GRID,),
    in_specs=[_row_spec(IN), _full_spec(IN, H), _full_spec(1, H),
              _row_spec(16), _full_spec(H, 16)],
    out_specs=[_row_spec(CW)] * C + [_row_spec(16)],
    out_shape=[jax.ShapeDtypeStruct((NP, CW), jnp.float32) for _ in range(C)]
    + [jax.ShapeDtypeStruct((NP, 16), jnp.float32)],
)

_tc_mid = pl.pallas_call(
    _tc_mid_body,
    grid=(GRID,),
    in_specs=[_row_spec(CW)] * (2 * C) + [_full_spec(H, 16), _row_spec(16)],
    out_specs=[_row_spec(CW)] * C + [_row_spec(16)],
    out_shape=[jax.ShapeDtypeStruct((NP, CW), jnp.float32) for _ in range(C)]
    + [jax.ShapeDtypeStruct((NP, 16), jnp.float32)],
)

_tc_final = pl.pallas_call(
    _tc_final_body,
    grid=(GRID,),
    in_specs=[_row_spec(CW)] * (2 * C) + [_full_spec(H, OUT), _full_spec(1, OUT)],
    out_specs=_row_spec(OUT),
    out_shape=jax.ShapeDtypeStruct((NP, OUT), jnp.float32),
)


def kernel(x, edge_index, W_in, b_in, att_l, att_r, W_out, b_out):
    loop = jnp.arange(N, dtype=jnp.int32)
    src = jnp.concatenate([edge_index[0], loop])
    dst = jnp.concatenate([edge_index[1], loop])
    pad = jnp.full((EP - E - N,), PAD_NODE, dtype=jnp.int32)
    src2d = jnp.concatenate([src, pad]).reshape(RB, EB)
    dst2d = jnp.concatenate([dst, pad]).reshape(RB, EB)
    xp = jnp.pad(x, ((0, NP - N), (0, 0)))
    att_big = [
        jnp.zeros((H, 16), jnp.float32)
        .at[:, 0].set(att_l[l]).at[:, 1].set(att_r[l])
        for l in range(L)
    ]
    b_in2 = b_in[None, :]
    b_out2 = b_out[None, :]

    deg16 = _sc_degree(dst2d)
    *h0c, t = _tc_prep(xp, W_in, b_in2, deg16, att_big[0])
    hc = list(h0c)
    oc = None
    for l in range(L):
        al1, ar1, di1 = t[:, 0], t[:, 1], t[:, 2]
        oc = _sc_spmm(src2d, dst2d, al1, ar1, di1, *hc)
        if l + 1 < L:
            *hc, t = _tc_mid(*oc, *h0c, att_big[l + 1], deg16)
    y = _tc_final(*oc, *h0c, W_out, b_out2)
    return y[:N]
